# initial kernel scaffold (unmeasured)
import jax
import jax.numpy as jnp
from jax import lax
from jax.experimental import pallas as pl
from jax.experimental.pallas import tpu as pltpu


def kernel(
    x,
):
    def body(*refs):
        pass

    out_shape = jax.ShapeDtypeStruct(..., jnp.float32)
    return pl.pallas_call(body, out_shape=out_shape)(...)


# baseline (device time: 124481 ns/iter reference)
import jax
import jax.numpy as jnp
from jax import lax
from jax.experimental import pallas as pl
from jax.experimental.pallas import tpu as pltpu

N_DEV = 32


def kernel(x):
    m, n_total = x.shape
    blk = n_total // N_DEV

    def body(x_ref, out_ref, send_sems, recv_sems):
        my_i = lax.axis_index("i")

        for s in range(1, N_DEV):
            j = (my_i + s) % N_DEV
            rdma = pltpu.make_async_remote_copy(
                src_ref=x_ref.at[:, pl.ds(j * blk, blk)],
                dst_ref=out_ref.at[pl.ds(my_i * m, m), :],
                send_sem=send_sems.at[j],
                recv_sem=recv_sems.at[my_i],
                device_id=j,
                device_id_type=pl.DeviceIdType.LOGICAL,
            )
            rdma.start()

        out_ref[pl.ds(my_i * m, m), :] = x_ref[:, pl.ds(my_i * blk, blk)]

        for s in range(1, N_DEV):
            k = (my_i - s) % N_DEV
            recv = pltpu.make_async_remote_copy(
                src_ref=x_ref.at[:, pl.ds(0, blk)],
                dst_ref=out_ref.at[pl.ds(k * m, m), :],
                send_sem=send_sems.at[k],
                recv_sem=recv_sems.at[k],
                device_id=k,
                device_id_type=pl.DeviceIdType.LOGICAL,
            )
            recv.wait_recv()

        for s in range(1, N_DEV):
            j = (my_i + s) % N_DEV
            send = pltpu.make_async_remote_copy(
                src_ref=x_ref.at[:, pl.ds(j * blk, blk)],
                dst_ref=out_ref.at[pl.ds(0, m), :],
                send_sem=send_sems.at[j],
                recv_sem=recv_sems.at[my_i],
                device_id=j,
                device_id_type=pl.DeviceIdType.LOGICAL,
            )
            send.wait_send()

    return pl.pallas_call(
        body,
        out_shape=jax.ShapeDtypeStruct((N_DEV * m, blk), x.dtype),
        in_specs=[pl.BlockSpec(memory_space=pltpu.VMEM)],
        out_specs=pl.BlockSpec(memory_space=pltpu.VMEM),
        scratch_shapes=[
            pltpu.SemaphoreType.DMA((N_DEV,)),
            pltpu.SemaphoreType.DMA((N_DEV,)),
        ],
    )(x)
